# baseline (speedup 1.0000x reference)
import jax
import jax.numpy as jnp
from jax import lax
from jax.experimental import pallas as pl
from jax.experimental.pallas import tpu as pltpu
from jax.experimental.pallas import tpu_sc as plsc

_NC = 2
_NS = 16
_L = 16
_BP = 128
_CB = 32
_CPTS = _CB * _BP
_CG = _CPTS // _L
_UG = 4
_UM = 8


def _i32(x):
    return jnp.asarray(x, dtype=jnp.int32)


def _make_sc_kernel(N, D, n_rows, HB):
    NW = _NC * _NS
    NCHF = N // _CPTS
    TP = N - NCHF * _CPTS
    NCH = NCHF + (1 if TP else 0)
    TG = TP // _L
    NBLK = -(-N // _BP)
    TBW = (NBLK - NCHF * _CB) * 4 * _BP
    OUTW = NBLK * 4 * _BP
    K1 = -(-NCHF // _NS)
    K2 = -(-NCH // NW)
    HSL = HB // _NS
    NB_OFF = (n_rows - 1) * _L

    def body(coords, pf, pi, binass_o, flat_o, hist_o,
             xbuf, ybuf, zbuf, fidx, bbuf, onesv, fidx_t, ones_t,
             pfv, piv, aminv, minv, zv, shist, smin):
        c = lax.axis_index("c")
        s = lax.axis_index("s")
        wid = s * _NC + c
        lanes = lax.iota(jnp.int32, _L)
        zero16 = jnp.zeros((_L,), jnp.int32)
        one16 = jnp.full((_L,), 1, jnp.int32)
        inf16 = jnp.full((_L,), jnp.inf, jnp.float32)
        bufs = (xbuf, ybuf, zbuf)

        pltpu.sync_copy(pf, pfv)
        pltpu.sync_copy(pi, piv)

        def _fill_ones(i, _):
            onesv[pl.ds(i * _L, _L)] = one16
            return 0
        lax.fori_loop(_i32(0), _i32(_CG), _fill_ones, 0)

        def _fill_ones_t(i, _):
            ones_t[pl.ds(i * _L, _L)] = one16
            return 0
        lax.fori_loop(_i32(0), _i32(TG), _fill_ones_t, 0)

        def _fill_z(i, _):
            zv[pl.ds(i * _L, _L)] = zero16
            return 0
        lax.fori_loop(_i32(0), _i32(HSL // _L), _fill_z, 0)
        pltpu.sync_copy(zv, shist.at[pl.ds(s * HSL, HSL)])

        def _min_chunk(k, carry):
            ch = jnp.minimum(s + _NS * k, _i32(NCHF - 1))
            for d in range(D):
                pltpu.sync_copy(coords.at[pl.ds(d * N + ch * _CPTS, _CPTS)],
                                bufs[d])

            def _g(g, mm):
                b = g * (_L * _UM)
                for u in range(_UM):
                    mm = tuple(
                        jnp.minimum(mm[d],
                                    bufs[d][pl.ds(b + u * _L, _L)])
                        for d in range(D))
                return mm
            return lax.fori_loop(_i32(0), _i32(_CG // _UM), _g, carry)
        with jax.named_scope("p1_min"):
            mins = lax.fori_loop(_i32(0), _i32(K1), _min_chunk, (inf16,) * D)
        if TP:
            for d in range(D):
                pltpu.sync_copy(coords.at[pl.ds(d * N + NCHF * _CPTS, TP)],
                                bufs[d].at[pl.ds(0, TP)])

            def _gt(g, mm):
                b = g * _L
                return tuple(jnp.minimum(mm[d], bufs[d][pl.ds(b, _L)])
                             for d in range(D))
            mins = lax.fori_loop(_i32(0), _i32(TG), _gt, mins)
        for d in range(D):
            minv[pl.ds(d * _L, _L)] = mins[d]
        pltpu.sync_copy(minv, smin.at[pl.ds(s * D * _L, D * _L)])
        plsc.subcore_barrier()
        pltpu.sync_copy(smin, aminv)

        def _red(t, mm):
            b = t * (D * _L)
            return tuple(jnp.minimum(mm[d], aminv[pl.ds(b + d * _L, _L)])
                         for d in range(D))
        gm = lax.fori_loop(_i32(0), _i32(_NS), _red, (inf16,) * D)
        dmin = [jnp.broadcast_to(jnp.min(gm[d]), (_L,)) for d in range(D)]

        bw = pfv[pl.ds(0, _L)]
        invbw = jnp.full((_L,), 1.0, jnp.float32) / bw
        nf = [pfv[pl.ds((1 + d) * _L, _L)] for d in range(D)]
        rsv = [piv[pl.ds(r * _L, _L)] for r in range(n_rows - 1)]
        nb = [piv[pl.ds(NB_OFF + d * _L, _L)] for d in range(D)]

        def _rowid(pt):
            r = zero16
            for rb in rsv:
                r = r + jnp.where(pt >= rb, one16, zero16)
            return r

        def _one_group(ptb0, fb, g, u, unroll, r_const):
            b = g * (_L * unroll) + u * _L
            ii = []
            for d in range(D):
                q = (bufs[d][pl.ds(b, _L)] - dmin[d]) * invbw
                q = jnp.minimum(q, nf[d])
                ii.append(q.astype(jnp.int32))
            if r_const is None:
                r = _rowid(ptb0 + b + lanes)
            else:
                r = r_const
            fl = r
            for d in range(D):
                fl = fl * nb[d] + ii[d]
            fb[pl.ds(b, _L)] = fl
            gg = b // _L
            base = (gg // 8) * (4 * _BP) + (gg % 8) * _L
            bbuf[pl.ds(base, _L)] = r
            for d in range(D):
                bbuf[pl.ds(base + (d + 1) * _BP, _L)] = ii[d]

        def _run_groups(ptb0, fb, nsteps, unroll, r_const):
            def _g(g, _c):
                for u in range(unroll):
                    _one_group(ptb0, fb, g, u, unroll, r_const)
                return 0
            lax.fori_loop(_i32(0), _i32(nsteps), _g, 0)

        def _chunk2(k, _):
            ch = wid + NW * k

            @pl.when(ch < NCHF)
            def _():
                for d in range(D):
                    pltpu.sync_copy(coords.at[pl.ds(d * N + ch * _CPTS, _CPTS)],
                                    bufs[d])
                ptb0 = ch * _CPTS
                _run_groups(ptb0, fidx, _CG // _UG, _UG, None)
                pltpu.sync_copy(fidx, flat_o.at[pl.ds(ch * _CPTS, _CPTS)])
                pltpu.sync_copy(bbuf, binass_o.at[pl.ds(ch * 4 * _CPTS,
                                                        4 * _CPTS)])
                pltpu.sync_copy(onesv, shist.at[fidx], add=True)

            if TP:
                @pl.when(ch == NCHF)
                def _():
                    for d in range(D):
                        pltpu.sync_copy(
                            coords.at[pl.ds(d * N + NCHF * _CPTS, TP)],
                            bufs[d].at[pl.ds(0, TP)])
                    ptb0 = _i32(NCHF * _CPTS)
                    _run_groups(ptb0, fidx_t, TG, 1, None)
                    pltpu.sync_copy(fidx_t, flat_o.at[pl.ds(NCHF * _CPTS, TP)])
                    pltpu.sync_copy(bbuf.at[pl.ds(0, TBW)],
                                    binass_o.at[pl.ds(NCHF * 4 * _CPTS, TBW)])
                    pltpu.sync_copy(ones_t, shist.at[fidx_t], add=True)
            return 0
        with jax.named_scope("p2_bins"):
            lax.fori_loop(_i32(0), _i32(K2), _chunk2, 0)

        plsc.subcore_barrier()
        with jax.named_scope("p3_hist_export"):
            pltpu.sync_copy(shist.at[pl.ds(s * HSL, HSL)], zv)
        pltpu.sync_copy(zv, hist_o.at[pl.ds(c * HB + s * HSL, HSL)])

    mesh = plsc.VectorSubcoreMesh(core_axis_name="c", subcore_axis_name="s")
    return pl.kernel(
        body,
        out_type=(
            jax.ShapeDtypeStruct((OUTW,), jnp.int32),
            jax.ShapeDtypeStruct((N,), jnp.int32),
            jax.ShapeDtypeStruct((_NC * HB,), jnp.int32),
        ),
        mesh=mesh,
        compiler_params=pltpu.CompilerParams(needs_layout_passes=False),
        scratch_types=[
            pltpu.VMEM((_CPTS,), jnp.float32),
            pltpu.VMEM((_CPTS,), jnp.float32),
            pltpu.VMEM((_CPTS,), jnp.float32),
            pltpu.VMEM((_CPTS,), jnp.int32),
            pltpu.VMEM((4 * _CPTS,), jnp.int32),
            pltpu.VMEM((_CPTS,), jnp.int32),
            pltpu.VMEM((max(TP, _L),), jnp.int32),
            pltpu.VMEM((max(TP, _L),), jnp.int32),
            pltpu.VMEM(((1 + D) * _L,), jnp.float32),
            pltpu.VMEM(((n_rows - 1 + D) * _L,), jnp.int32),
            pltpu.VMEM((_NS * 3 * _L,), jnp.float32),
            pltpu.VMEM((3 * _L,), jnp.float32),
            pltpu.VMEM((HSL,), jnp.int32),
            pltpu.VMEM_SHARED((HB,), jnp.int32),
            pltpu.VMEM_SHARED((_NS * 3 * _L,), jnp.float32),
        ],
    )


def _tc_add(hist2, HB):
    def bdy(h_ref, o_ref):
        o_ref[...] = h_ref[0] + h_ref[1]
    out = pl.pallas_call(
        bdy,
        out_shape=jax.ShapeDtypeStruct((HB // 128, 128), jnp.int32),
    )(hist2.reshape(_NC, HB // 128, 128))
    return out.reshape(HB)


def kernel(coordinates, row_splits, bin_width, nbins, return_all):
    N, D = coordinates.shape
    n_rows = row_splits.shape[0] - 1
    HB = n_rows * 24 ** D
    NBLK = -(-N // _BP)

    coords_planes = coordinates.T.reshape(N * D)
    rs32 = row_splits.astype(jnp.int32)
    pi = jnp.concatenate([
        jnp.broadcast_to(rs32[1:n_rows, None], (n_rows - 1, _L)),
        jnp.broadcast_to(nbins[:, None], (D, _L)),
    ], axis=0).reshape(-1)
    pf = jnp.concatenate([
        jnp.broadcast_to(bin_width[:, None], (1, _L)),
        jnp.broadcast_to((nbins[:, None] - 1).astype(jnp.float32), (D, _L)),
    ], axis=0).reshape(-1)

    fn = _make_sc_kernel(N, D, n_rows, HB)
    binass_blocks, flat, hist2 = fn(coords_planes, pf, pi)
    binass = (binass_blocks.reshape(NBLK, D + 1, _BP)
              .transpose(0, 2, 1).reshape(NBLK * _BP, D + 1)[:N])
    n_per_bin = _tc_add(hist2, HB)
    return binass, flat, nbins, bin_width, n_per_bin


# speedup vs baseline: 1.3893x; 1.3893x over previous
import jax
import jax.numpy as jnp
from jax import lax
from jax.experimental import pallas as pl
from jax.experimental.pallas import tpu as pltpu
from jax.experimental.pallas import tpu_sc as plsc

_NC = 2
_NS = 16
_L = 16
_BP = 128
_CB = 32
_CPTS = _CB * _BP
_CG = _CPTS // _L
_UG = 4
_UM = 8


def _i32(x):
    return jnp.asarray(x, dtype=jnp.int32)


def _make_sc_kernel(N, D, n_rows, HB):
    NW = _NC * _NS
    NCHF = N // _CPTS
    TP = N - NCHF * _CPTS
    TG = TP // _L
    NBLK = -(-N // _BP)
    TBW = (NBLK - NCHF * _CB) * 4 * _BP
    OUTW = NBLK * 4 * _BP
    K1 = -(-NCHF // _NS)
    K2 = -(-(NCHF + (1 if TP else 0)) // NW)
    WLIM = NCHF - NW * (K2 - 1)
    HSL = HB // _NS
    NB_OFF = (n_rows - 1) * _L
    assert NW * (K2 - 2) + NW - 1 < NCHF and 0 <= WLIM <= NW
    assert K1 % 2 == 0

    def body(coords, pf, pi, binass_o, flat_o, hist_o,
             xa, ya, za, xb, yb, zb, fxa, fxb, bba, bbb, onesv,
             fidx_t, ones_t, pfv, piv, aminv, minv, zv, shist, smin,
             sin_a, sin_b, sout_a, sout_b, shs_a, shs_b):
        c = lax.axis_index("c")
        s = lax.axis_index("s")
        wid = s * _NC + c
        lanes = lax.iota(jnp.int32, _L)
        zero16 = jnp.zeros((_L,), jnp.int32)
        one16 = jnp.full((_L,), 1, jnp.int32)
        inf16 = jnp.full((_L,), jnp.inf, jnp.float32)
        bufs = ((xa, ya, za), (xb, yb, zb))
        fx = (fxa, fxb)
        bb = (bba, bbb)
        sin = (sin_a, sin_b)
        sout = (sout_a, sout_b)
        shs = (shs_a, shs_b)

        pltpu.sync_copy(pf, pfv)
        pltpu.sync_copy(pi, piv)

        def _fill_ones(i, _):
            onesv[pl.ds(i * _L, _L)] = one16
            return 0
        lax.fori_loop(_i32(0), _i32(_CG), _fill_ones, 0)

        def _fill_ones_t(i, _):
            ones_t[pl.ds(i * _L, _L)] = one16
            return 0
        lax.fori_loop(_i32(0), _i32(TG), _fill_ones_t, 0)

        def _fill_z(i, _):
            zv[pl.ds(i * _L, _L)] = zero16
            return 0
        lax.fori_loop(_i32(0), _i32(HSL // _L), _fill_z, 0)
        pltpu.sync_copy(zv, shist.at[pl.ds(s * HSL, HSL)])

        def _p1ch(k):
            return jnp.minimum(s + _NS * k, _i32(NCHF - 1))

        def _p1fire(ch, p):
            off = pl.multiple_of(ch * _CPTS, _CPTS)
            for d in range(D):
                pltpu.async_copy(coords.at[pl.ds(d * N + off, _CPTS)],
                                 bufs[p][d], sin[p])

        def _p1drain(p):
            for d in range(D):
                pltpu.make_async_copy(coords.at[pl.ds(0, _CPTS)],
                                      bufs[p][d], sin[p]).wait()

        def _minred(p, mm):
            def _g(g, m2):
                b = g * (_L * _UM)
                for u in range(_UM):
                    m2 = tuple(
                        jnp.minimum(m2[d],
                                    bufs[p][d][pl.ds(b + u * _L, _L)])
                        for d in range(D))
                return m2
            return lax.fori_loop(_i32(0), _i32(_CG // _UM), _g, mm)

        _p1fire(_p1ch(_i32(0)), 0)

        def _min_pair(j, mm):
            _p1fire(_p1ch(2 * j + 1), 1)
            _p1drain(0)
            mm = _minred(0, mm)
            _p1fire(_p1ch(2 * j + 2), 0)
            _p1drain(1)
            return _minred(1, mm)
        mins = lax.fori_loop(_i32(0), _i32(K1 // 2), _min_pair, (inf16,) * D)
        _p1drain(0)
        if TP:
            for d in range(D):
                pltpu.sync_copy(coords.at[pl.ds(d * N + NCHF * _CPTS, TP)],
                                bufs[0][d].at[pl.ds(0, TP)])

            def _gt(g, mm):
                b = g * _L
                return tuple(jnp.minimum(mm[d], bufs[0][d][pl.ds(b, _L)])
                             for d in range(D))
            mins = lax.fori_loop(_i32(0), _i32(TG), _gt, mins)
        for d in range(D):
            minv[pl.ds(d * _L, _L)] = mins[d]
        pltpu.sync_copy(minv, smin.at[pl.ds(s * D * _L, D * _L)])
        plsc.subcore_barrier()
        pltpu.sync_copy(smin, aminv)

        def _red(t, mm):
            b = t * (D * _L)
            return tuple(jnp.minimum(mm[d], aminv[pl.ds(b + d * _L, _L)])
                         for d in range(D))
        gm = lax.fori_loop(_i32(0), _i32(_NS), _red, (inf16,) * D)
        dmin = [jnp.broadcast_to(jnp.min(gm[d]), (_L,)) for d in range(D)]

        bw = pfv[pl.ds(0, _L)]
        invbw = jnp.full((_L,), 1.0, jnp.float32) / bw
        nf = [pfv[pl.ds((1 + d) * _L, _L)] for d in range(D)]
        rsv = [piv[pl.ds(r * _L, _L)] for r in range(n_rows - 1)]
        nb = [piv[pl.ds(NB_OFF + d * _L, _L)] for d in range(D)]

        def _rowid(pt):
            r = zero16
            for rb in rsv:
                r = r + jnp.where(pt >= rb, one16, zero16)
            return r

        def _one_group(ptb0, bufp, fb, bbp, g, u, unroll):
            b = g * (_L * unroll) + u * _L
            ii = []
            for d in range(D):
                q = (bufp[d][pl.ds(b, _L)] - dmin[d]) * invbw
                q = jnp.minimum(q, nf[d])
                ii.append(q.astype(jnp.int32))
            r = _rowid(ptb0 + b + lanes)
            fl = r
            for d in range(D):
                fl = fl * nb[d] + ii[d]
            fb[pl.ds(b, _L)] = fl
            gg = b // _L
            base = (gg // 8) * (4 * _BP) + (gg % 8) * _L
            bbp[pl.ds(base, _L)] = r
            for d in range(D):
                bbp[pl.ds(base + (d + 1) * _BP, _L)] = ii[d]

        def _run_groups(ptb0, bufp, fb, bbp, nsteps, unroll):
            def _g(g, _c):
                for u in range(unroll):
                    _one_group(ptb0, bufp, fb, bbp, g, u, unroll)
                return 0
            lax.fori_loop(_i32(0), _i32(nsteps), _g, 0)

        def _fire_in(ch, p):
            off = pl.multiple_of(ch * _CPTS, _CPTS)
            for d in range(D):
                pltpu.async_copy(coords.at[pl.ds(d * N + off, _CPTS)],
                                 bufs[p][d], sin[p])

        def _drain_in(p):
            for d in range(D):
                pltpu.make_async_copy(coords.at[pl.ds(0, _CPTS)],
                                      bufs[p][d], sin[p]).wait()

        def _fire_out(ch, p):
            off = pl.multiple_of(ch * _CPTS, _CPTS)
            off4 = pl.multiple_of(ch * (4 * _CPTS), 4 * _CPTS)
            pltpu.async_copy(fx[p], flat_o.at[pl.ds(off, _CPTS)], sout[p])
            pltpu.async_copy(bb[p], binass_o.at[pl.ds(off4, 4 * _CPTS)],
                             sout[p])
            pltpu.async_copy(onesv, shist.at[fx[p]], shs[p], add=True)

        def _drain_out(p):
            pltpu.make_async_copy(fx[p], flat_o.at[pl.ds(0, _CPTS)],
                                  sout[p]).wait()
            pltpu.make_async_copy(bb[p], binass_o.at[pl.ds(0, 4 * _CPTS)],
                                  sout[p]).wait()
            pltpu.make_async_copy(onesv, shist.at[fx[p]], shs[p]).wait()

        def _compute(ch, p):
            _run_groups(ch * _CPTS, bufs[p], fx[p], bb[p], _CG // _UG, _UG)

        _fire_in(wid, 0)
        for k in range(K2 - 1):
            p = k % 2
            if k + 1 < K2 - 1:
                _fire_in(wid + _i32(NW * (k + 1)), 1 - p)
            else:
                @pl.when(wid < WLIM)
                def _(k=k, p=p):
                    _fire_in(wid + _i32(NW * (k + 1)), 1 - p)
            _drain_in(p)
            if k >= 2:
                _drain_out(p)
            _compute(wid + _i32(NW * k), p)
            _fire_out(wid + _i32(NW * k), p)
        _drain_out((K2 - 3) % 2)
        _drain_out((K2 - 2) % 2)

        plast = (K2 - 1) % 2

        @pl.when(wid < WLIM)
        def _():
            _drain_in(plast)
            ch = wid + _i32(NW * (K2 - 1))
            _compute(ch, plast)
            _fire_out(ch, plast)
            _drain_out(plast)

        if TP:
            @pl.when(wid == WLIM)
            def _():
                for d in range(D):
                    pltpu.sync_copy(
                        coords.at[pl.ds(d * N + NCHF * _CPTS, TP)],
                        bufs[0][d].at[pl.ds(0, TP)])
                _run_groups(_i32(NCHF * _CPTS), bufs[0], fidx_t, bb[0], TG, 1)
                pltpu.sync_copy(fidx_t, flat_o.at[pl.ds(NCHF * _CPTS, TP)])
                pltpu.sync_copy(bb[0].at[pl.ds(0, TBW)],
                                binass_o.at[pl.ds(NCHF * 4 * _CPTS, TBW)])
                pltpu.sync_copy(ones_t, shist.at[fidx_t], add=True)

        plsc.subcore_barrier()
        pltpu.sync_copy(shist.at[pl.ds(s * HSL, HSL)], zv)
        pltpu.sync_copy(zv, hist_o.at[pl.ds(c * HB + s * HSL, HSL)])

    mesh = plsc.VectorSubcoreMesh(core_axis_name="c", subcore_axis_name="s")
    return pl.kernel(
        body,
        out_type=(
            jax.ShapeDtypeStruct((OUTW,), jnp.int32),
            jax.ShapeDtypeStruct((N,), jnp.int32),
            jax.ShapeDtypeStruct((_NC * HB,), jnp.int32),
        ),
        mesh=mesh,
        compiler_params=pltpu.CompilerParams(needs_layout_passes=False),
        scratch_types=[
            pltpu.VMEM((_CPTS,), jnp.float32),
            pltpu.VMEM((_CPTS,), jnp.float32),
            pltpu.VMEM((_CPTS,), jnp.float32),
            pltpu.VMEM((_CPTS,), jnp.float32),
            pltpu.VMEM((_CPTS,), jnp.float32),
            pltpu.VMEM((_CPTS,), jnp.float32),
            pltpu.VMEM((_CPTS,), jnp.int32),
            pltpu.VMEM((_CPTS,), jnp.int32),
            pltpu.VMEM((4 * _CPTS,), jnp.int32),
            pltpu.VMEM((4 * _CPTS,), jnp.int32),
            pltpu.VMEM((_CPTS,), jnp.int32),
            pltpu.VMEM((max(TP, _L),), jnp.int32),
            pltpu.VMEM((max(TP, _L),), jnp.int32),
            pltpu.VMEM(((1 + D) * _L,), jnp.float32),
            pltpu.VMEM(((n_rows - 1 + D) * _L,), jnp.int32),
            pltpu.VMEM((_NS * 3 * _L,), jnp.float32),
            pltpu.VMEM((3 * _L,), jnp.float32),
            pltpu.VMEM((HSL,), jnp.int32),
            pltpu.VMEM_SHARED((HB,), jnp.int32),
            pltpu.VMEM_SHARED((_NS * 3 * _L,), jnp.float32),
            pltpu.SemaphoreType.DMA,
            pltpu.SemaphoreType.DMA,
            pltpu.SemaphoreType.DMA,
            pltpu.SemaphoreType.DMA,
            pltpu.SemaphoreType.DMA,
            pltpu.SemaphoreType.DMA,
        ],
    )


def _tc_add(hist2, HB):
    def bdy(h_ref, o_ref):
        o_ref[...] = h_ref[0] + h_ref[1]
    out = pl.pallas_call(
        bdy,
        out_shape=jax.ShapeDtypeStruct((HB // 128, 128), jnp.int32),
    )(hist2.reshape(_NC, HB // 128, 128))
    return out.reshape(HB)


def kernel(coordinates, row_splits, bin_width, nbins, return_all):
    N, D = coordinates.shape
    n_rows = row_splits.shape[0] - 1
    HB = n_rows * 24 ** D
    NBLK = -(-N // _BP)

    coords_planes = coordinates.T.reshape(N * D)
    rs32 = row_splits.astype(jnp.int32)
    pi = jnp.concatenate([
        jnp.broadcast_to(rs32[1:n_rows, None], (n_rows - 1, _L)),
        jnp.broadcast_to(nbins[:, None], (D, _L)),
    ], axis=0).reshape(-1)
    pf = jnp.concatenate([
        jnp.broadcast_to(bin_width[:, None], (1, _L)),
        jnp.broadcast_to((nbins[:, None] - 1).astype(jnp.float32), (D, _L)),
    ], axis=0).reshape(-1)

    fn = _make_sc_kernel(N, D, n_rows, HB)
    binass_blocks, flat, hist2 = fn(coords_planes, pf, pi)
    binass = (binass_blocks.reshape(NBLK, D + 1, _BP)
              .transpose(0, 2, 1).reshape(NBLK * _BP, D + 1)[:N])
    n_per_bin = _tc_add(hist2, HB)
    return binass, flat, nbins, bin_width, n_per_bin


# speedup vs baseline: 1.4015x; 1.0088x over previous
import jax
import jax.numpy as jnp
from jax import lax
from jax.experimental import pallas as pl
from jax.experimental.pallas import tpu as pltpu
from jax.experimental.pallas import tpu_sc as plsc

_NC = 2
_NS = 16
_L = 16
_BP = 128
_CB = 32
_CPTS = _CB * _BP
_CG = _CPTS // _L
_UG = 8
_UM = 8


def _i32(x):
    return jnp.asarray(x, dtype=jnp.int32)


def _make_sc_kernel(N, D, n_rows, HB):
    NW = _NC * _NS
    NCHF = N // _CPTS
    TP = N - NCHF * _CPTS
    TG = TP // _L
    NBLK = -(-N // _BP)
    TBW = (NBLK - NCHF * _CB) * 4 * _BP
    OUTW = NBLK * 4 * _BP
    K1 = -(-NCHF // _NS)
    K2 = -(-(NCHF + (1 if TP else 0)) // NW)
    WLIM = NCHF - NW * (K2 - 1)
    HSL = HB // _NS
    NB_OFF = (n_rows - 1) * _L
    assert NW * (K2 - 2) + NW - 1 < NCHF and 0 <= WLIM <= NW
    assert K1 % 2 == 0

    def body(coords, pf, pi, binass_o, flat_o, hist_o,
             xa, ya, za, xb, yb, zb, fxa, fxb, bba, bbb, onesv,
             fidx_t, ones_t, pfv, piv, aminv, minv, zv, shist, smin,
             sin_a, sin_b, sout_a, sout_b, shs_a, shs_b):
        c = lax.axis_index("c")
        s = lax.axis_index("s")
        wid = s * _NC + c
        lanes = lax.iota(jnp.int32, _L)
        zero16 = jnp.zeros((_L,), jnp.int32)
        one16 = jnp.full((_L,), 1, jnp.int32)
        inf16 = jnp.full((_L,), jnp.inf, jnp.float32)
        bufs = ((xa, ya, za), (xb, yb, zb))
        fx = (fxa, fxb)
        bb = (bba, bbb)
        sin = (sin_a, sin_b)
        sout = (sout_a, sout_b)
        shs = (shs_a, shs_b)

        pltpu.sync_copy(pf, pfv)
        pltpu.sync_copy(pi, piv)

        def _fill_ones(i, _):
            onesv[pl.ds(i * _L, _L)] = one16
            return 0
        lax.fori_loop(_i32(0), _i32(_CG), _fill_ones, 0)

        def _fill_ones_t(i, _):
            ones_t[pl.ds(i * _L, _L)] = one16
            return 0
        lax.fori_loop(_i32(0), _i32(TG), _fill_ones_t, 0)

        def _fill_z(i, _):
            zv[pl.ds(i * _L, _L)] = zero16
            return 0
        lax.fori_loop(_i32(0), _i32(HSL // _L), _fill_z, 0)
        pltpu.sync_copy(zv, shist.at[pl.ds(s * HSL, HSL)])

        def _p1ch(k):
            return jnp.minimum(s + _NS * k, _i32(NCHF - 1))

        def _p1fire(ch, p):
            off = pl.multiple_of(ch * _CPTS, _CPTS)
            for d in range(D):
                pltpu.async_copy(coords.at[pl.ds(d * N + off, _CPTS)],
                                 bufs[p][d], sin[p])

        def _p1drain(p):
            for d in range(D):
                pltpu.make_async_copy(coords.at[pl.ds(0, _CPTS)],
                                      bufs[p][d], sin[p]).wait()

        def _minred(p, mm):
            def _g(g, m2):
                b = g * (_L * _UM)
                for u in range(_UM):
                    m2 = tuple(
                        jnp.minimum(m2[d],
                                    bufs[p][d][pl.ds(b + u * _L, _L)])
                        for d in range(D))
                return m2
            return lax.fori_loop(_i32(0), _i32(_CG // _UM), _g, mm)

        _p1fire(_p1ch(_i32(0)), 0)

        def _min_pair(j, mm):
            _p1fire(_p1ch(2 * j + 1), 1)
            _p1drain(0)
            mm = _minred(0, mm)
            _p1fire(_p1ch(2 * j + 2), 0)
            _p1drain(1)
            return _minred(1, mm)
        mins = lax.fori_loop(_i32(0), _i32(K1 // 2), _min_pair, (inf16,) * D)
        _p1drain(0)
        if TP:
            for d in range(D):
                pltpu.sync_copy(coords.at[pl.ds(d * N + NCHF * _CPTS, TP)],
                                bufs[0][d].at[pl.ds(0, TP)])

            def _gt(g, mm):
                b = g * _L
                return tuple(jnp.minimum(mm[d], bufs[0][d][pl.ds(b, _L)])
                             for d in range(D))
            mins = lax.fori_loop(_i32(0), _i32(TG), _gt, mins)
        for d in range(D):
            minv[pl.ds(d * _L, _L)] = mins[d]
        pltpu.sync_copy(minv, smin.at[pl.ds(s * D * _L, D * _L)])
        plsc.subcore_barrier()
        pltpu.sync_copy(smin, aminv)

        def _red(t, mm):
            b = t * (D * _L)
            return tuple(jnp.minimum(mm[d], aminv[pl.ds(b + d * _L, _L)])
                         for d in range(D))
        gm = lax.fori_loop(_i32(0), _i32(_NS), _red, (inf16,) * D)
        dmin = [jnp.broadcast_to(jnp.min(gm[d]), (_L,)) for d in range(D)]

        bw = pfv[pl.ds(0, _L)]
        invbw = jnp.full((_L,), 1.0, jnp.float32) / bw
        nf = [pfv[pl.ds((1 + d) * _L, _L)] for d in range(D)]
        rsv = [piv[pl.ds(r * _L, _L)] for r in range(n_rows - 1)]
        nb = [piv[pl.ds(NB_OFF + d * _L, _L)] for d in range(D)]

        def _rowid(pt):
            r = zero16
            for rb in rsv:
                r = r + jnp.where(pt >= rb, one16, zero16)
            return r

        def _one_group(ptb0, bufp, fb, bbp, g, u, unroll):
            b = g * (_L * unroll) + u * _L
            ii = []
            for d in range(D):
                q = (bufp[d][pl.ds(b, _L)] - dmin[d]) * invbw
                q = jnp.minimum(q, nf[d])
                ii.append(q.astype(jnp.int32))
            r = _rowid(ptb0 + b + lanes)
            fl = r
            for d in range(D):
                fl = fl * nb[d] + ii[d]
            fb[pl.ds(b, _L)] = fl
            gg = b // _L
            base = (gg // 8) * (4 * _BP) + (gg % 8) * _L
            bbp[pl.ds(base, _L)] = r
            for d in range(D):
                bbp[pl.ds(base + (d + 1) * _BP, _L)] = ii[d]

        def _run_groups(ptb0, bufp, fb, bbp, nsteps, unroll):
            def _g(g, _c):
                for u in range(unroll):
                    _one_group(ptb0, bufp, fb, bbp, g, u, unroll)
                return 0
            lax.fori_loop(_i32(0), _i32(nsteps), _g, 0)

        def _fire_in(ch, p):
            off = pl.multiple_of(ch * _CPTS, _CPTS)
            for d in range(D):
                pltpu.async_copy(coords.at[pl.ds(d * N + off, _CPTS)],
                                 bufs[p][d], sin[p])

        def _drain_in(p):
            for d in range(D):
                pltpu.make_async_copy(coords.at[pl.ds(0, _CPTS)],
                                      bufs[p][d], sin[p]).wait()

        def _fire_out(ch, p):
            off = pl.multiple_of(ch * _CPTS, _CPTS)
            off4 = pl.multiple_of(ch * (4 * _CPTS), 4 * _CPTS)
            pltpu.async_copy(fx[p], flat_o.at[pl.ds(off, _CPTS)], sout[p])
            pltpu.async_copy(bb[p], binass_o.at[pl.ds(off4, 4 * _CPTS)],
                             sout[p])
            pltpu.async_copy(onesv, shist.at[fx[p]], shs[p], add=True)

        def _drain_out(p):
            pltpu.make_async_copy(fx[p], flat_o.at[pl.ds(0, _CPTS)],
                                  sout[p]).wait()
            pltpu.make_async_copy(bb[p], binass_o.at[pl.ds(0, 4 * _CPTS)],
                                  sout[p]).wait()
            pltpu.make_async_copy(onesv, shist.at[fx[p]], shs[p]).wait()

        def _compute(ch, p):
            _run_groups(ch * _CPTS, bufs[p], fx[p], bb[p], _CG // _UG, _UG)

        _fire_in(wid, 0)
        for k in range(K2 - 1):
            p = k % 2
            if k + 1 < K2 - 1:
                _fire_in(wid + _i32(NW * (k + 1)), 1 - p)
            else:
                @pl.when(wid < WLIM)
                def _(k=k, p=p):
                    _fire_in(wid + _i32(NW * (k + 1)), 1 - p)
            _drain_in(p)
            if k >= 2:
                _drain_out(p)
            _compute(wid + _i32(NW * k), p)
            _fire_out(wid + _i32(NW * k), p)
        _drain_out((K2 - 3) % 2)
        _drain_out((K2 - 2) % 2)

        plast = (K2 - 1) % 2

        @pl.when(wid < WLIM)
        def _():
            _drain_in(plast)
            ch = wid + _i32(NW * (K2 - 1))
            _compute(ch, plast)
            _fire_out(ch, plast)
            _drain_out(plast)

        if TP:
            @pl.when(wid == WLIM)
            def _():
                for d in range(D):
                    pltpu.sync_copy(
                        coords.at[pl.ds(d * N + NCHF * _CPTS, TP)],
                        bufs[0][d].at[pl.ds(0, TP)])
                _run_groups(_i32(NCHF * _CPTS), bufs[0], fidx_t, bb[0], TG, 1)
                pltpu.sync_copy(fidx_t, flat_o.at[pl.ds(NCHF * _CPTS, TP)])
                pltpu.sync_copy(bb[0].at[pl.ds(0, TBW)],
                                binass_o.at[pl.ds(NCHF * 4 * _CPTS, TBW)])
                pltpu.sync_copy(ones_t, shist.at[fidx_t], add=True)

        plsc.subcore_barrier()
        pltpu.sync_copy(shist.at[pl.ds(s * HSL, HSL)], zv)
        pltpu.sync_copy(zv, hist_o.at[pl.ds(c * HB + s * HSL, HSL)])

    mesh = plsc.VectorSubcoreMesh(core_axis_name="c", subcore_axis_name="s")
    return pl.kernel(
        body,
        out_type=(
            jax.ShapeDtypeStruct((OUTW,), jnp.int32),
            jax.ShapeDtypeStruct((N,), jnp.int32),
            jax.ShapeDtypeStruct((_NC * HB,), jnp.int32),
        ),
        mesh=mesh,
        compiler_params=pltpu.CompilerParams(needs_layout_passes=False),
        scratch_types=[
            pltpu.VMEM((_CPTS,), jnp.float32),
            pltpu.VMEM((_CPTS,), jnp.float32),
            pltpu.VMEM((_CPTS,), jnp.float32),
            pltpu.VMEM((_CPTS,), jnp.float32),
            pltpu.VMEM((_CPTS,), jnp.float32),
            pltpu.VMEM((_CPTS,), jnp.float32),
            pltpu.VMEM((_CPTS,), jnp.int32),
            pltpu.VMEM((_CPTS,), jnp.int32),
            pltpu.VMEM((4 * _CPTS,), jnp.int32),
            pltpu.VMEM((4 * _CPTS,), jnp.int32),
            pltpu.VMEM((_CPTS,), jnp.int32),
            pltpu.VMEM((max(TP, _L),), jnp.int32),
            pltpu.VMEM((max(TP, _L),), jnp.int32),
            pltpu.VMEM(((1 + D) * _L,), jnp.float32),
            pltpu.VMEM(((n_rows - 1 + D) * _L,), jnp.int32),
            pltpu.VMEM((_NS * 3 * _L,), jnp.float32),
            pltpu.VMEM((3 * _L,), jnp.float32),
            pltpu.VMEM((HSL,), jnp.int32),
            pltpu.VMEM_SHARED((HB,), jnp.int32),
            pltpu.VMEM_SHARED((_NS * 3 * _L,), jnp.float32),
            pltpu.SemaphoreType.DMA,
            pltpu.SemaphoreType.DMA,
            pltpu.SemaphoreType.DMA,
            pltpu.SemaphoreType.DMA,
            pltpu.SemaphoreType.DMA,
            pltpu.SemaphoreType.DMA,
        ],
    )


def _tc_add(hist2, HB):
    def bdy(h_ref, o_ref):
        o_ref[...] = h_ref[0] + h_ref[1]
    out = pl.pallas_call(
        bdy,
        out_shape=jax.ShapeDtypeStruct((HB // 128, 128), jnp.int32),
    )(hist2.reshape(_NC, HB // 128, 128))
    return out.reshape(HB)


def kernel(coordinates, row_splits, bin_width, nbins, return_all):
    N, D = coordinates.shape
    n_rows = row_splits.shape[0] - 1
    HB = n_rows * 24 ** D
    NBLK = -(-N // _BP)

    coords_planes = coordinates.T.reshape(N * D)
    rs32 = row_splits.astype(jnp.int32)
    pi = jnp.concatenate([
        jnp.broadcast_to(rs32[1:n_rows, None], (n_rows - 1, _L)),
        jnp.broadcast_to(nbins[:, None], (D, _L)),
    ], axis=0).reshape(-1)
    pf = jnp.concatenate([
        jnp.broadcast_to(bin_width[:, None], (1, _L)),
        jnp.broadcast_to((nbins[:, None] - 1).astype(jnp.float32), (D, _L)),
    ], axis=0).reshape(-1)

    fn = _make_sc_kernel(N, D, n_rows, HB)
    binass_blocks, flat, hist2 = fn(coords_planes, pf, pi)
    binass = (binass_blocks.reshape(NBLK, D + 1, _BP)
              .transpose(0, 2, 1).reshape(NBLK * _BP, D + 1)[:N])
    n_per_bin = _tc_add(hist2, HB)
    return binass, flat, nbins, bin_width, n_per_bin


# speedup vs baseline: 1.4086x; 1.0051x over previous
import jax
import jax.numpy as jnp
from jax import lax
from jax.experimental import pallas as pl
from jax.experimental.pallas import tpu as pltpu
from jax.experimental.pallas import tpu_sc as plsc

_NC = 2
_NS = 16
_L = 16
_BP = 128
_CB = 32
_CPTS = _CB * _BP
_CG = _CPTS // _L
_UG = 8
_UM = 8


def _i32(x):
    return jnp.asarray(x, dtype=jnp.int32)


def _make_sc_kernel(N, D, n_rows, HB):
    NW = _NC * _NS
    NCHF = N // _CPTS
    TP = N - NCHF * _CPTS
    TG = TP // _L
    NBLK = -(-N // _BP)
    TBW = (NBLK - NCHF * _CB) * 4 * _BP
    OUTW = NBLK * 4 * _BP
    K1 = -(-NCHF // _NS)
    K2 = -(-(NCHF + (1 if TP else 0)) // NW)
    WLIM = NCHF - NW * (K2 - 1)
    HSL = HB // _NS
    NB_OFF = (n_rows - 1) * _L
    assert NW * (K2 - 2) + NW - 1 < NCHF and 0 <= WLIM <= NW
    assert K1 % 2 == 0

    def body(coords, pf, pi, binass_o, flat_o, hist_o,
             xa, ya, za, xb, yb, zb, fxa, fxb, bba, bbb, onesv,
             fidx_t, ones_t, pfv, piv, aminv, minv, zv, shist, smin,
             sin_a, sin_b, sout_a, sout_b, shs_a, shs_b):
        c = lax.axis_index("c")
        s = lax.axis_index("s")
        wid = s * _NC + c
        lanes = lax.iota(jnp.int32, _L)
        zero16 = jnp.zeros((_L,), jnp.int32)
        one16 = jnp.full((_L,), 1, jnp.int32)
        inf16 = jnp.full((_L,), jnp.inf, jnp.float32)
        bufs = ((xa, ya, za), (xb, yb, zb))
        fx = (fxa, fxb)
        bb = (bba, bbb)
        sin = (sin_a, sin_b)
        sout = (sout_a, sout_b)
        shs = (shs_a, shs_b)

        def _p1ch(k):
            return jnp.minimum(s + _NS * k, _i32(NCHF - 1))

        def _p1fire(ch, p):
            off = pl.multiple_of(ch * _CPTS, _CPTS)
            for d in range(D):
                pltpu.async_copy(coords.at[pl.ds(d * N + off, _CPTS)],
                                 bufs[p][d], sin[p])

        _p1fire(_p1ch(_i32(0)), 0)
        pltpu.sync_copy(pf, pfv)
        pltpu.sync_copy(pi, piv)

        def _fill_ones(i, _):
            onesv[pl.ds(i * _L, _L)] = one16
            return 0
        lax.fori_loop(_i32(0), _i32(_CG), _fill_ones, 0)

        def _fill_ones_t(i, _):
            ones_t[pl.ds(i * _L, _L)] = one16
            return 0
        lax.fori_loop(_i32(0), _i32(TG), _fill_ones_t, 0)

        def _fill_z(i, _):
            zv[pl.ds(i * _L, _L)] = zero16
            return 0
        lax.fori_loop(_i32(0), _i32(HSL // _L), _fill_z, 0)
        pltpu.sync_copy(zv, shist.at[pl.ds(s * HSL, HSL)])

        def _p1drain(p):
            for d in range(D):
                pltpu.make_async_copy(coords.at[pl.ds(0, _CPTS)],
                                      bufs[p][d], sin[p]).wait()

        def _minred(p, mm):
            def _g(g, m2):
                b = g * (_L * _UM)
                for u in range(_UM):
                    m2 = tuple(
                        jnp.minimum(m2[d],
                                    bufs[p][d][pl.ds(b + u * _L, _L)])
                        for d in range(D))
                return m2
            return lax.fori_loop(_i32(0), _i32(_CG // _UM), _g, mm)

        def _min_pair(j, mm):
            _p1fire(_p1ch(2 * j + 1), 1)
            _p1drain(0)
            mm = _minred(0, mm)
            _p1fire(_p1ch(2 * j + 2), 0)
            _p1drain(1)
            return _minred(1, mm)
        mins = lax.fori_loop(_i32(0), _i32(K1 // 2), _min_pair, (inf16,) * D)
        _p1drain(0)
        if TP:
            for d in range(D):
                pltpu.sync_copy(coords.at[pl.ds(d * N + NCHF * _CPTS, TP)],
                                bufs[0][d].at[pl.ds(0, TP)])

            def _gt(g, mm):
                b = g * _L
                return tuple(jnp.minimum(mm[d], bufs[0][d][pl.ds(b, _L)])
                             for d in range(D))
            mins = lax.fori_loop(_i32(0), _i32(TG), _gt, mins)
        for d in range(D):
            minv[pl.ds(d * _L, _L)] = mins[d]
        pltpu.sync_copy(minv, smin.at[pl.ds(s * D * _L, D * _L)])
        plsc.subcore_barrier()
        pltpu.sync_copy(smin, aminv)

        def _red(t, mm):
            b = t * (D * _L)
            return tuple(jnp.minimum(mm[d], aminv[pl.ds(b + d * _L, _L)])
                         for d in range(D))
        gm = lax.fori_loop(_i32(0), _i32(_NS), _red, (inf16,) * D)
        dmin = [jnp.broadcast_to(jnp.min(gm[d]), (_L,)) for d in range(D)]

        bw = pfv[pl.ds(0, _L)]
        invbw = jnp.full((_L,), 1.0, jnp.float32) / bw
        nf = [pfv[pl.ds((1 + d) * _L, _L)] for d in range(D)]
        rsv = [piv[pl.ds(r * _L, _L)] for r in range(n_rows - 1)]
        nb = [piv[pl.ds(NB_OFF + d * _L, _L)] for d in range(D)]

        def _rowid(pt):
            r = zero16
            for rb in rsv:
                r = r + jnp.where(pt >= rb, one16, zero16)
            return r

        def _one_group(ptb0, bufp, fb, bbp, g, u, unroll):
            b = g * (_L * unroll) + u * _L
            ii = []
            for d in range(D):
                q = (bufp[d][pl.ds(b, _L)] - dmin[d]) * invbw
                q = jnp.minimum(q, nf[d])
                ii.append(q.astype(jnp.int32))
            r = _rowid(ptb0 + b + lanes)
            fl = r
            for d in range(D):
                fl = fl * nb[d] + ii[d]
            fb[pl.ds(b, _L)] = fl
            gg = b // _L
            base = (gg // 8) * (4 * _BP) + (gg % 8) * _L
            bbp[pl.ds(base, _L)] = r
            for d in range(D):
                bbp[pl.ds(base + (d + 1) * _BP, _L)] = ii[d]

        def _run_groups(ptb0, bufp, fb, bbp, nsteps, unroll):
            def _g(g, _c):
                for u in range(unroll):
                    _one_group(ptb0, bufp, fb, bbp, g, u, unroll)
                return 0
            lax.fori_loop(_i32(0), _i32(nsteps), _g, 0)

        def _fire_in(ch, p):
            off = pl.multiple_of(ch * _CPTS, _CPTS)
            for d in range(D):
                pltpu.async_copy(coords.at[pl.ds(d * N + off, _CPTS)],
                                 bufs[p][d], sin[p])

        def _drain_in(p):
            for d in range(D):
                pltpu.make_async_copy(coords.at[pl.ds(0, _CPTS)],
                                      bufs[p][d], sin[p]).wait()

        def _fire_out(ch, p):
            off = pl.multiple_of(ch * _CPTS, _CPTS)
            off4 = pl.multiple_of(ch * (4 * _CPTS), 4 * _CPTS)
            pltpu.async_copy(fx[p], flat_o.at[pl.ds(off, _CPTS)], sout[p])
            pltpu.async_copy(bb[p], binass_o.at[pl.ds(off4, 4 * _CPTS)],
                             sout[p])
            pltpu.async_copy(onesv, shist.at[fx[p]], shs[p], add=True)

        def _drain_out(p):
            pltpu.make_async_copy(fx[p], flat_o.at[pl.ds(0, _CPTS)],
                                  sout[p]).wait()
            pltpu.make_async_copy(bb[p], binass_o.at[pl.ds(0, 4 * _CPTS)],
                                  sout[p]).wait()
            pltpu.make_async_copy(onesv, shist.at[fx[p]], shs[p]).wait()

        def _compute(ch, p):
            _run_groups(ch * _CPTS, bufs[p], fx[p], bb[p], _CG // _UG, _UG)

        _fire_in(wid, 0)
        for k in range(K2 - 1):
            p = k % 2
            if k + 1 < K2 - 1:
                _fire_in(wid + _i32(NW * (k + 1)), 1 - p)
            else:
                @pl.when(wid < WLIM)
                def _(k=k, p=p):
                    _fire_in(wid + _i32(NW * (k + 1)), 1 - p)
            _drain_in(p)
            if k >= 2:
                _drain_out(p)
            _compute(wid + _i32(NW * k), p)
            _fire_out(wid + _i32(NW * k), p)
        _drain_out((K2 - 3) % 2)
        _drain_out((K2 - 2) % 2)

        plast = (K2 - 1) % 2

        @pl.when(wid < WLIM)
        def _():
            _drain_in(plast)
            ch = wid + _i32(NW * (K2 - 1))
            _compute(ch, plast)
            _fire_out(ch, plast)
            _drain_out(plast)

        if TP:
            @pl.when(wid == WLIM)
            def _():
                for d in range(D):
                    pltpu.sync_copy(
                        coords.at[pl.ds(d * N + NCHF * _CPTS, TP)],
                        bufs[0][d].at[pl.ds(0, TP)])
                _run_groups(_i32(NCHF * _CPTS), bufs[0], fidx_t, bb[0], TG, 1)
                pltpu.sync_copy(fidx_t, flat_o.at[pl.ds(NCHF * _CPTS, TP)])
                pltpu.sync_copy(bb[0].at[pl.ds(0, TBW)],
                                binass_o.at[pl.ds(NCHF * 4 * _CPTS, TBW)])
                pltpu.sync_copy(ones_t, shist.at[fidx_t], add=True)

        plsc.subcore_barrier()
        pltpu.sync_copy(shist.at[pl.ds(s * HSL, HSL)], zv)
        pltpu.sync_copy(zv, hist_o.at[pl.ds(c * HB + s * HSL, HSL)])

    mesh = plsc.VectorSubcoreMesh(core_axis_name="c", subcore_axis_name="s")
    return pl.kernel(
        body,
        out_type=(
            jax.ShapeDtypeStruct((OUTW,), jnp.int32),
            jax.ShapeDtypeStruct((N,), jnp.int32),
            jax.ShapeDtypeStruct((_NC * HB,), jnp.int32),
        ),
        mesh=mesh,
        compiler_params=pltpu.CompilerParams(needs_layout_passes=False),
        scratch_types=[
            pltpu.VMEM((_CPTS,), jnp.float32),
            pltpu.VMEM((_CPTS,), jnp.float32),
            pltpu.VMEM((_CPTS,), jnp.float32),
            pltpu.VMEM((_CPTS,), jnp.float32),
            pltpu.VMEM((_CPTS,), jnp.float32),
            pltpu.VMEM((_CPTS,), jnp.float32),
            pltpu.VMEM((_CPTS,), jnp.int32),
            pltpu.VMEM((_CPTS,), jnp.int32),
            pltpu.VMEM((4 * _CPTS,), jnp.int32),
            pltpu.VMEM((4 * _CPTS,), jnp.int32),
            pltpu.VMEM((_CPTS,), jnp.int32),
            pltpu.VMEM((max(TP, _L),), jnp.int32),
            pltpu.VMEM((max(TP, _L),), jnp.int32),
            pltpu.VMEM(((1 + D) * _L,), jnp.float32),
            pltpu.VMEM(((n_rows - 1 + D) * _L,), jnp.int32),
            pltpu.VMEM((_NS * 3 * _L,), jnp.float32),
            pltpu.VMEM((3 * _L,), jnp.float32),
            pltpu.VMEM((HSL,), jnp.int32),
            pltpu.VMEM_SHARED((HB,), jnp.int32),
            pltpu.VMEM_SHARED((_NS * 3 * _L,), jnp.float32),
            pltpu.SemaphoreType.DMA,
            pltpu.SemaphoreType.DMA,
            pltpu.SemaphoreType.DMA,
            pltpu.SemaphoreType.DMA,
            pltpu.SemaphoreType.DMA,
            pltpu.SemaphoreType.DMA,
        ],
    )


def _tc_add(hist2, HB):
    def bdy(h_ref, o_ref):
        o_ref[...] = h_ref[0] + h_ref[1]
    out = pl.pallas_call(
        bdy,
        out_shape=jax.ShapeDtypeStruct((HB // 128, 128), jnp.int32),
    )(hist2.reshape(_NC, HB // 128, 128))
    return out.reshape(HB)


def kernel(coordinates, row_splits, bin_width, nbins, return_all):
    N, D = coordinates.shape
    n_rows = row_splits.shape[0] - 1
    HB = n_rows * 24 ** D
    NBLK = -(-N // _BP)

    coords_planes = coordinates.T.reshape(N * D)
    rs32 = row_splits.astype(jnp.int32)
    pi = jnp.concatenate([
        jnp.broadcast_to(rs32[1:n_rows, None], (n_rows - 1, _L)),
        jnp.broadcast_to(nbins[:, None], (D, _L)),
    ], axis=0).reshape(-1)
    pf = jnp.concatenate([
        jnp.broadcast_to(bin_width[:, None], (1, _L)),
        jnp.broadcast_to((nbins[:, None] - 1).astype(jnp.float32), (D, _L)),
    ], axis=0).reshape(-1)

    fn = _make_sc_kernel(N, D, n_rows, HB)
    binass_blocks, flat, hist2 = fn(coords_planes, pf, pi)
    binass = (binass_blocks.reshape(NBLK, D + 1, _BP)
              .transpose(0, 2, 1).reshape(NBLK * _BP, D + 1)[:N])
    n_per_bin = _tc_add(hist2, HB)
    return binass, flat, nbins, bin_width, n_per_bin
